# Initial kernel scaffold; baseline (speedup 1.0000x reference)
import jax
import jax.numpy as jnp
from jax.experimental import pallas as pl


def kernel(sequence, c_sequence, token_table, content_table, pe):
    raise NotImplementedError("write your pallas kernel here")


# speedup vs baseline: 6.0804x; 6.0804x over previous
import functools

import jax
import jax.numpy as jnp
from jax import lax
from jax.experimental import pallas as pl
from jax.experimental.pallas import tpu as pltpu
from jax.experimental.pallas import tpu_sc as plsc

E = 128
LSEQ = 200
B = 1024
K = 4
N = B * LSEQ
NW = 32
TPW = N // NW
T = 128
NCHUNK = TPW // T
EG = E // 16


def _body(seq_hbm, cidx_hbm, tok_tab, cont_tab, pe_hbm, out_hbm,
          pe_v, idx_v, cidx_v, acc_v, t0_v, t1_v, t2_v, t3_v,
          sem_g, sem_c):
    c = lax.axis_index("c")
    s = lax.axis_index("s")
    wid = s * 2 + c

    pltpu.sync_copy(pe_hbm, pe_v)

    tmps = (t0_v, t1_v, t2_v, t3_v)

    def chunk_body(n, _):
        base = wid * TPW + n * T
        pltpu.sync_copy(seq_hbm.at[pl.ds(base, T)], idx_v)
        for k in range(K):
            pltpu.sync_copy(cidx_hbm.at[k, pl.ds(base, T)], cidx_v.at[k])
        g = pltpu.async_copy(tok_tab.at[idx_v], acc_v, sem_g)
        cps = [pltpu.async_copy(cont_tab.at[cidx_v.at[k]], tmps[k], sem_c)
               for k in range(K)]
        g.wait()
        for cp in cps:
            cp.wait()

        l0 = (n * T) % LSEQ

        def row(i, _):
            li = l0 + i
            li = jnp.where(li >= LSEQ, li - LSEQ, li)
            for j in range(EG):
                sl = pl.ds(j * 16, 16)
                v = t0_v[i, sl] + t1_v[i, sl]
                w = t2_v[i, sl] + t3_v[i, sl]
                v = v + w + pe_v[li, sl]
                plsc.addupdate(acc_v.at[i, sl], v)
            return 0

        lax.fori_loop(0, T, row, 0)

        pltpu.sync_copy(acc_v, out_hbm.at[pl.ds(base, T)])
        return 0

    lax.fori_loop(0, NCHUNK, chunk_body, 0)


@functools.partial(jax.jit, static_argnames=())
def _run(seq_flat, cidx, token_table, content_table, pe200):
    kern = pl.kernel(
        _body,
        out_type=jax.ShapeDtypeStruct((N, E), jnp.float32),
        mesh=plsc.VectorSubcoreMesh(core_axis_name="c", subcore_axis_name="s"),
        scratch_types=[
            pltpu.VMEM((LSEQ, E), jnp.float32),
            pltpu.VMEM((T,), jnp.int32),
            pltpu.VMEM((K, T), jnp.int32),
            pltpu.VMEM((T, E), jnp.float32),
            pltpu.VMEM((T, E), jnp.float32),
            pltpu.VMEM((T, E), jnp.float32),
            pltpu.VMEM((T, E), jnp.float32),
            pltpu.VMEM((T, E), jnp.float32),
            pltpu.SemaphoreType.DMA,
            pltpu.SemaphoreType.DMA,
        ],
    )
    return kern(seq_flat, cidx, token_table, content_table, pe200)


def kernel(sequence, c_sequence, token_table, content_table, pe):
    seq_flat = sequence.reshape(N).astype(jnp.int32)
    cidx = c_sequence.reshape(N, K).T.astype(jnp.int32)
    pe200 = pe[0, :LSEQ]
    out = _run(seq_flat, cidx, token_table, content_table, pe200)
    return out.reshape(B, LSEQ, E)


# speedup vs baseline: 8.2849x; 1.3626x over previous
import functools

import numpy as np

import jax
import jax.numpy as jnp
from jax import lax
from jax.experimental import pallas as pl
from jax.experimental.pallas import tpu as pltpu
from jax.experimental.pallas import tpu_sc as plsc

E = 128
EH = E // 2
LSEQ = 200
B = 1024
K = 4
KP = K + 1
N = B * LSEQ
NW = 32
TPW = N // NW
T = 32
NCHUNK = TPW // T
NITER = NCHUNK // 4

def _body(idxc_hbm, tok_tab, cont_tab, pe_hbm, out_hbm,
          pe_v, i0, i1, i2, i3, a0, a1, a2, a3, m0, m1, m2, m3,
          gs0, gs1, gs2, gs3, os0, os1, os2, os3, is0, is1, is2, is3):
    c = lax.axis_index("c")
    s = lax.axis_index("s")
    wid = s * 2 + c

    idxs = (i0, i1, i2, i3)
    accs = (a0, a1, a2, a3)
    tmps = (m0, m1, m2, m3)
    semG = (gs0, gs1, gs2, gs3)
    semO = (os0, os1, os2, os3)
    semI = (is0, is1, is2, is3)

    pltpu.sync_copy(pe_hbm, pe_v)

    def idx_copy(slot, n):
        return pltpu.make_async_copy(idxc_hbm.at[wid, n], idxs[slot], semI[slot])

    def gathers(slot):
        cps = [pltpu.make_async_copy(tok_tab.at[idxs[slot].at[0]], accs[slot],
                                     semG[slot])]
        for k in range(K):
            cps.append(pltpu.make_async_copy(cont_tab.at[idxs[slot].at[k + 1]],
                                             tmps[slot].at[k], semG[slot]))
        return cps

    def out_copy(slot, base):
        return pltpu.make_async_copy(accs[slot], out_hbm.at[pl.ds(base, T)],
                                     semO[slot])

    for j in range(4):
        idx_copy(j, j).start()
    for j in range(2):
        idx_copy(j, j).wait()
        for cp in gathers(j):
            cp.start()

    def compute(slot, n):
        acc = accs[slot]
        tm = tmps[slot]
        l0 = lax.rem(n * T, LSEQ)

        def row(i):
            li = l0 + i
            li = jnp.where(li >= LSEQ, li - LSEQ, li)
            for g in range(8):
                sl = pl.ds(16 * g, 16)
                v = (tm[0, i, sl] + tm[1, i, sl]) + (tm[2, i, sl] + tm[3, i, sl])
                v = v + pe_v[li, sl]
                plsc.addupdate(acc.at[i, sl], v)

        def rowpair(t, _):
            row(2 * t)
            row(2 * t + 1)
            return 0

        lax.fori_loop(0, T // 2, rowpair, 0)

    def chunk(u, m):
        n = 4 * m + u
        base = wid * TPW + n * T
        for cp in gathers(u):
            cp.wait()
        compute(u, n)
        out_copy(u, base).start()

        r = (u + 2) % 4

        def refill():
            idx_copy(r, n + 2).wait()
            for cp in gathers(r):
                cp.start()

        def issue_idx():
            idx_copy(u, n + 4).start()

        if u < 2:
            @pl.when(m > 0)
            def _():
                out_copy(r, base).wait()
            refill()

            @pl.when(m < NITER - 1)
            def _():
                issue_idx()
        else:
            @pl.when(m < NITER - 1)
            def _():
                out_copy(r, base).wait()
                refill()
                issue_idx()

    def body(m, _):
        for u in range(4):
            chunk(u, m)
        return 0

    lax.fori_loop(0, NITER, body, 0)

    for u in range(4):
        out_copy(u, wid * TPW).wait()


@jax.jit
def _run(idxc, token_table, content_table, pe200):
    kern = pl.kernel(
        _body,
        out_type=jax.ShapeDtypeStruct((N, E), jnp.float32),
        mesh=plsc.VectorSubcoreMesh(core_axis_name="c", subcore_axis_name="s"),
        scratch_types=(
            [pltpu.VMEM((LSEQ, E), jnp.float32)]
            + [pltpu.VMEM((KP, T), jnp.int32)] * 4
            + [pltpu.VMEM((T, E), jnp.float32)] * 4
            + [pltpu.VMEM((K, T, E), jnp.float32)] * 4
            + [pltpu.SemaphoreType.DMA] * 12
        ),
    )
    return kern(idxc, token_table, content_table, pe200)


def kernel(sequence, c_sequence, token_table, content_table, pe):
    tok = sequence.astype(jnp.int32).reshape(NW, NCHUNK, 1, T)
    con = (c_sequence.astype(jnp.int32)
           .reshape(NW, NCHUNK, T, K).transpose(0, 1, 3, 2))
    idxc = jnp.concatenate([tok, con], axis=2)
    out = _run(idxc, token_table, content_table, pe[0, :LSEQ])
    return out.reshape(B, LSEQ, E)


# speedup vs baseline: 11.1117x; 1.3412x over previous
import functools

import numpy as np

import jax
import jax.numpy as jnp
from jax import lax
from jax.experimental import pallas as pl
from jax.experimental.pallas import tpu as pltpu
from jax.experimental.pallas import tpu_sc as plsc

E = 128
LSEQ = 200
B = 1024
K = 4
KP = K + 1
N = B * LSEQ
NW = 32
TPW = N // NW
T = 64
NCHUNK = TPW // T
NITER = NCHUNK // 4

_PERM = (np.arange(4)[:, None] * 32
         + np.stack([np.arange(16), np.arange(16) + 16], 1).reshape(32)[None, :]
         ).reshape(128)


def _pack_table(tab):
    t = tab[:, _PERM].astype(jnp.bfloat16)
    return jax.lax.bitcast_convert_type(t.reshape(-1, 64, 2), jnp.int32)


def _body(idxc_hbm, tok_tab, cont_tab, pe_hbm, out_hbm,
          pe_v, i0, i1, i2, i3, a0, a1, a2, a3, m0, m1, m2, m3,
          gs0, gs1, gs2, gs3, os0, os1, os2, os3, is0, is1, is2, is3):
    c = lax.axis_index("c")
    s = lax.axis_index("s")
    wid = s * 2 + c

    idxs = (i0, i1, i2, i3)
    accs = (a0, a1, a2, a3)
    tmps = (m0, m1, m2, m3)
    semG = (gs0, gs1, gs2, gs3)
    semO = (os0, os1, os2, os3)
    semI = (is0, is1, is2, is3)

    pltpu.sync_copy(pe_hbm, pe_v)

    def idx_copy(slot, n):
        return pltpu.make_async_copy(idxc_hbm.at[wid, n], idxs[slot], semI[slot])

    def gathers(slot):
        cps = [pltpu.make_async_copy(tok_tab.at[idxs[slot].at[0]], accs[slot],
                                     semG[slot])]
        for k in range(K):
            cps.append(pltpu.make_async_copy(
                cont_tab.at[idxs[slot].at[k + 1]],
                tmps[slot].at[k], semG[slot]))
        return cps

    def out_copy(slot, base):
        return pltpu.make_async_copy(accs[slot], out_hbm.at[pl.ds(base, T)],
                                     semO[slot])

    for j in range(4):
        idx_copy(j, j).start()
    for j in range(2):
        idx_copy(j, j).wait()
        for cp in gathers(j):
            cp.start()

    def compute(slot, n):
        acc = accs[slot]
        tm = tmps[slot]
        l0 = lax.rem(n * T, LSEQ)

        def unpk(w):
            a = lax.bitcast_convert_type(w << 16, jnp.float32)
            b = lax.bitcast_convert_type(w & jnp.int32(-65536), jnp.float32)
            return a, b

        def row(i):
            li = l0 + i
            li = jnp.where(li >= LSEQ, li - LSEQ, li)
            for dg in range(4):
                wsl = pl.ds(16 * dg, 16)
                va = None
                vb = None
                for k in range(K):
                    a, b = unpk(tm[k, i, wsl])
                    va = a if va is None else va + a
                    vb = b if vb is None else vb + b
                ap, bp = unpk(pe_v[li, wsl])
                va = va + ap
                vb = vb + bp
                plsc.addupdate(acc.at[i, pl.ds(32 * dg, 16)], va)
                plsc.addupdate(acc.at[i, pl.ds(32 * dg + 16, 16)], vb)

        def rowpair(t, _):
            row(2 * t)
            row(2 * t + 1)
            return 0

        lax.fori_loop(0, T // 2, rowpair, 0)

    def chunk(u, m):
        n = 4 * m + u
        base = wid * TPW + n * T
        for cp in gathers(u):
            cp.wait()
        compute(u, n)
        out_copy(u, base).start()

        r = (u + 2) % 4

        def refill():
            idx_copy(r, n + 2).wait()
            for cp in gathers(r):
                cp.start()

        def issue_idx():
            idx_copy(u, n + 4).start()

        if u < 2:
            @pl.when(m > 0)
            def _():
                out_copy(r, base).wait()
            refill()

            @pl.when(m < NITER - 1)
            def _():
                issue_idx()
        else:
            @pl.when(m < NITER - 1)
            def _():
                out_copy(r, base).wait()
                refill()
                issue_idx()

    def body(m, _):
        for u in range(4):
            chunk(u, m)
        return 0

    lax.fori_loop(0, NITER, body, 0)

    for u in range(4):
        out_copy(u, wid * TPW).wait()


@jax.jit
def _run(idxc, token_table, cont_packed, pe_packed):
    kern = pl.kernel(
        _body,
        out_type=jax.ShapeDtypeStruct((N, E), jnp.float32),
        mesh=plsc.VectorSubcoreMesh(core_axis_name="c", subcore_axis_name="s"),
        compiler_params=pltpu.CompilerParams(use_tc_tiling_on_sc=False),
        scratch_types=(
            [pltpu.VMEM((LSEQ, E // 2), jnp.int32)]
            + [pltpu.VMEM((KP, T), jnp.int32)] * 4
            + [pltpu.VMEM((T, E), jnp.float32)] * 4
            + [pltpu.VMEM((K, T, E // 2), jnp.int32)] * 4
            + [pltpu.SemaphoreType.DMA] * 12
        ),
    )
    return kern(idxc, token_table, cont_packed, pe_packed)


def kernel(sequence, c_sequence, token_table, content_table, pe):
    tok = sequence.astype(jnp.int32).reshape(NW, NCHUNK, 1, T)
    con = (c_sequence.astype(jnp.int32)
           .reshape(NW, NCHUNK, T, K).transpose(0, 1, 3, 2))
    idxc = jnp.concatenate([tok, con], axis=2)
    cont_packed = _pack_table(content_table)
    pe_packed = _pack_table(pe[0, :LSEQ])
    out = _run(idxc, token_table, cont_packed, pe_packed)
    return out.reshape(B, LSEQ, E)
